# Initial kernel scaffold; baseline (speedup 1.0000x reference)
import jax
import jax.numpy as jnp
from jax.experimental import pallas as pl


def kernel(node_x, edge_x, edge_src, edge_dst, params):
    raise NotImplementedError("write your pallas kernel here")


# speedup vs baseline: 1.0186x; 1.0186x over previous
import functools

import jax
import jax.numpy as jnp
from jax import lax
from jax.experimental import pallas as pl
from jax.experimental.pallas import tpu as pltpu

N = 10000
E = 160000
D = 128
P = 20000
NF = 1200
NR = 30

_NEG = -1e30


def _erf(x):
    s = jnp.sign(x)
    a = jnp.abs(x)
    t = 1.0 / (1.0 + 0.3275911 * a)
    poly = t * (0.254829592 + t * (-0.284496736 + t * (1.421413741 +
           t * (-1.453152027 + t * 1.061405429))))
    return s * (1.0 - poly * jnp.exp(-a * a))


def _gelu(x):
    return 0.5 * x * (1.0 + _erf(x * 0.7071067811865476))


def _tables_body(emb, kwn, kbn, qwn, qbn, vwn, vbn, art, mrt, mri,
                 kwe, kbe, aro, vwe, vbe, mro, awe, abe, gew, rwt,
                 scal,
                 q_tab, krt_tab, vrt_tab, kro_tab, vro_tab,
                 hidA2_tab, embB_tab, eA, eB, roleA, roleB):
    x = emb[...]
    a_e = scal[0, 0]
    ct = scal[0, 1]
    co = scal[0, 2]
    q_tab[...] = x @ qwn[...] + qbn[...]
    krt_tab[...] = (x @ (kwn[...] @ art[...]) + kbn[...] @ art[...]) * ct
    vrt_tab[...] = x @ (vwn[...] @ mrt[...]) + vbn[...] @ mrt[...]
    kro_tab[...] = (x @ (kwe[...] @ aro[...]) + kbe[...] @ aro[...]) * co
    vro_tab[...] = x @ (vwe[...] @ mro[...]) + vbe[...] @ mro[...]
    vrin = x @ (vwn[...] @ mri[...]) + vbn[...] @ mri[...]
    hidA = _gelu(vrin) @ awe[...] + abe[...]
    hidA2 = a_e * hidA
    embB = (1.0 - a_e) * x
    hidA2_tab[...] = hidA2
    embB_tab[...] = embB
    eA[...] = hidA2 @ gew[...]
    eB[...] = embB @ gew[...]
    roleA[...] = hidA2 @ rwt[...]
    roleB[...] = embB @ rwt[...]


def _make_tables(emb, prm, a_e, rwt_pad):
    bp = 2000
    grid = (P // bp,)
    scal = jnp.stack([a_e,
                      prm['p_rel_true'] / jnp.sqrt(jnp.float32(D)),
                      prm['p_rel_out'] / jnp.sqrt(jnp.float32(D))]).reshape(1, 3)

    def rep(shape):
        return pl.BlockSpec(shape, lambda i: (0,) * len(shape))

    dd = rep((D, D))
    db = rep((1, D))
    in_specs = [pl.BlockSpec((bp, D), lambda i: (i, 0)),
                dd, db, dd, db, dd, db, dd, dd, dd,
                dd, db, dd, dd, db, dd, dd, db, rep((D, 1)), rep((D, 32)),
                rep((1, 3))]
    out_specs = [pl.BlockSpec((bp, D), lambda i: (i, 0))] * 7 + \
                [pl.BlockSpec((bp, 1), lambda i: (i, 0))] * 2 + \
                [pl.BlockSpec((bp, 32), lambda i: (i, 0))] * 2
    fd = jax.ShapeDtypeStruct((P, D), jnp.float32)
    f1 = jax.ShapeDtypeStruct((P, 1), jnp.float32)
    f32s = jax.ShapeDtypeStruct((P, 32), jnp.float32)
    out_shape = [fd] * 7 + [f1, f1, f32s, f32s]
    args = (emb,
            prm['k_w_node'], prm['k_b_node'].reshape(1, D),
            prm['q_w_node'], prm['q_b_node'].reshape(1, D),
            prm['v_w_node'], prm['v_b_node'].reshape(1, D),
            prm['a_rel_true'], prm['m_rel_true'], prm['m_rel_in'],
            prm['k_w_edge'], prm['k_b_edge'].reshape(1, D),
            prm['a_rel_out'],
            prm['v_w_edge'], prm['v_b_edge'].reshape(1, D),
            prm['m_rel_out'],
            prm['a_w_edge'], prm['a_b_edge'].reshape(1, D),
            prm['gat_e_w'], rwt_pad, scal)
    return pl.pallas_call(
        _tables_body, grid=grid, in_specs=in_specs, out_specs=out_specs,
        out_shape=out_shape)(*args)


def _node_body(aggp, x_node, awn, abn, glw, grw, scal, xl, xr, gn):
    agg = aggp[0] + aggp[1]
    a_n = scal[0, 0]
    o = _gelu(agg) @ awn[...] + abn[...]
    hid = a_n * o + (1.0 - a_n) * x_node[...]
    xl[...] = hid @ glw[...] + scal[0, 1]
    xr[...] = hid @ grw[...] + scal[0, 2]
    m = jnp.max(hid, axis=0, keepdims=True)
    ez = jnp.exp(hid - m)
    gn[...] = jnp.sum(ez * hid, axis=0, keepdims=True) / jnp.sum(ez, axis=0, keepdims=True)


def _node_stage(agg_partials, x_node, prm):
    scal = jnp.stack([jax.nn.sigmoid(prm['skip_node']),
                      prm['gat_l_b'][0], prm['gat_r_b'][0]]).reshape(1, 3)
    return pl.pallas_call(
        _node_body,
        out_shape=[jax.ShapeDtypeStruct((N, 1), jnp.float32),
                   jax.ShapeDtypeStruct((N, 1), jnp.float32),
                   jax.ShapeDtypeStruct((1, D), jnp.float32)],
    )(agg_partials, x_node, prm['a_w_node'], prm['a_b_node'].reshape(1, D),
      prm['gat_l_w'], prm['gat_r_w'], scal)


def _root_body(gatp, cntp, xl, xr, scal, root_preds, amax):
    g = jnp.sum(gatp[...], axis=0)
    cnt = jnp.sum(cntp[...], axis=0, keepdims=True)
    att = scal[0, 0]
    bias = scal[0, 1]
    xlr = xl[...].reshape(1, N)
    xrr = xr[...].reshape(1, N)
    loop_eproj = g[0:1] / jnp.maximum(cnt, 1.0)
    z = xlr + xrr + loop_eproj
    s_self = jnp.maximum(z, 0.2 * z) * att
    es = jnp.exp(s_self)
    den = g[1:2] + es
    num = g[2:3] + es * xlr
    root = num / (den + 1e-16) + bias
    m = jnp.max(root, axis=1, keepdims=True)
    e = jnp.exp(root - m)
    lse = jnp.log(jnp.sum(e, axis=1, keepdims=True))
    root_preds[...] = root - m - lse
    idx = lax.broadcasted_iota(jnp.int32, (1, N), 1)
    amax[...] = jnp.min(jnp.where(root == m, idx, N), axis=1, keepdims=True)


def _root_stage(gat_partials, cnt_partials, xl, xr, prm):
    scal = jnp.stack([prm['gat_att'][0], prm['gat_bias'][0]]).reshape(1, 2)
    return pl.pallas_call(
        _root_body,
        out_shape=[jax.ShapeDtypeStruct((1, N), jnp.float32),
                   jax.ShapeDtypeStruct((1, 1), jnp.int32)],
    )(gat_partials, cnt_partials, xl, xr, scal)


def _frame_body(gn, gep, fw, fb, rwb, rb, frame, const32):
    num = jnp.sum(gep[...], axis=0)
    ge = num[0:1] / num[1:2]
    grep = jnp.concatenate([gn[...], ge], axis=1)
    f = grep @ fw[...] + fb[...]
    m = jnp.max(f, axis=1, keepdims=True)
    lse = jnp.log(jnp.sum(jnp.exp(f - m), axis=1, keepdims=True))
    frame[...] = f - m - lse
    const32[...] = gn[...] @ rwb[...] + rb[...]


def _frame_stage(gn, ge_partials, prm, rwb_pad, rb_pad):
    return pl.pallas_call(
        _frame_body,
        out_shape=[jax.ShapeDtypeStruct((1, NF), jnp.float32),
                   jax.ShapeDtypeStruct((1, 32), jnp.float32)],
    )(gn, ge_partials, prm['frame_w'], prm['frame_b'].reshape(1, NF),
      rwb_pad, rb_pad)


def _role_body(role_pre, src, amax, const32, out):
    r = role_pre[...] + const32[...]
    keep = src[...] == amax[0, 0]
    r = jnp.where(keep, r, 0.0)
    lane = lax.broadcasted_iota(jnp.int32, r.shape, 1)
    valid = lane < NR
    rm = jnp.where(valid, r, _NEG)
    m = jnp.max(rm, axis=1, keepdims=True)
    e = jnp.where(valid, jnp.exp(r - m), 0.0)
    lse = jnp.log(jnp.sum(e, axis=1, keepdims=True))
    out[...] = r - m - lse


def _role_stage(role_pre, edge_src, amax, const32):
    be = 2000
    grid = (E // be,)
    return pl.pallas_call(
        _role_body, grid=grid,
        in_specs=[pl.BlockSpec((be, 32), lambda i: (i, 0)),
                  pl.BlockSpec((be, 1), lambda i: (i, 0)),
                  pl.BlockSpec((1, 1), lambda i: (0, 0)),
                  pl.BlockSpec((1, 32), lambda i: (0, 0))],
        out_specs=pl.BlockSpec((be, 32), lambda i: (i, 0)),
        out_shape=jax.ShapeDtypeStruct((E, 32), jnp.float32),
    )(role_pre, edge_src.reshape(E, 1), amax, const32)


def kernel(node_x, edge_x, edge_src, edge_dst, params):
    prm = params
    emb = prm['pred_emb']
    a_e = jax.nn.sigmoid(prm['skip_edge'])
    rwt_pad = jnp.pad(prm['role_w'][:D], ((0, 0), (0, 32 - NR)))
    rwb_pad = jnp.pad(prm['role_w'][D:], ((0, 0), (0, 32 - NR)))
    rb_pad = jnp.pad(prm['role_b'], (0, 32 - NR)).reshape(1, 32)

    (q_tab, krt_tab, vrt_tab, kro_tab, vro_tab,
     hidA2_tab, embB_tab, eA, eB, roleA, roleB) = _make_tables(emb, prm, a_e, rwt_pad)

    cs = node_x[edge_src]
    cd = node_x[edge_dst]
    ce = edge_x
    x_node = emb[node_x]

    qg = q_tab[cd]
    e_t = jnp.exp(jnp.sum(krt_tab[cs] * qg, axis=-1))
    e_o = jnp.exp(jnp.sum(kro_tab[ce] * qg, axis=-1))
    den_t = jax.ops.segment_sum(e_t, edge_dst, num_segments=N)
    den_o = jax.ops.segment_sum(e_o, edge_dst, num_segments=N)
    cnt_partials = jax.ops.segment_sum(jnp.ones((E,), jnp.float32), edge_dst,
                                       num_segments=N).reshape(1, N)

    alpha_t = e_t / (den_t[edge_dst] + 1e-16)
    alpha_o = e_o / (den_o[edge_dst] + 1e-16)
    agg = jax.ops.segment_sum(
        vrt_tab[cs] * alpha_t[:, None] + vro_tab[ce] * alpha_o[:, None],
        edge_dst, num_segments=N)
    agg_partials = jnp.stack([agg, jnp.zeros_like(agg)])

    z = hidA2_tab[cs] + embB_tab[ce]
    ez = jnp.exp(z)
    ge_partials = jnp.stack([jnp.sum(ez * z, axis=0),
                             jnp.sum(ez, axis=0)]).reshape(1, 2, D)

    xl, xr, gn = _node_stage(agg_partials, x_node, prm)

    xlf = xl[:, 0]
    xrf = xr[:, 0]
    eproj = eA[:, 0][cs] + eB[:, 0][ce]
    zs = xlf[edge_src] + xrf[edge_dst] + eproj
    s_e = jnp.maximum(zs, 0.2 * zs) * prm['gat_att'][0]
    es_e = jnp.exp(s_e)
    loopsum = jax.ops.segment_sum(eproj, edge_dst, num_segments=N)
    den_g = jax.ops.segment_sum(es_e, edge_dst, num_segments=N)
    num_g = jax.ops.segment_sum(es_e * xlf[edge_src], edge_dst, num_segments=N)
    gat_partials = jnp.stack([loopsum, den_g, num_g]).reshape(1, 3, N)

    root_preds2, amax = _root_stage(gat_partials, cnt_partials, xl, xr, prm)
    frame2, const32 = _frame_stage(gn, ge_partials, prm, rwb_pad, rb_pad)

    role_pre = roleA[cs] + roleB[ce]

    role32 = _role_stage(role_pre, edge_src, amax, const32)

    root_preds = root_preds2.reshape(N)
    frame_preds = frame2.reshape(NF)
    role_preds = role32[:, :NR]
    return ((root_preds, frame_preds), role_preds)


# speedup vs baseline: 5.3369x; 5.2393x over previous
import functools

import jax
import jax.numpy as jnp
from jax import lax
from jax.experimental import pallas as pl
from jax.experimental.pallas import tpu as pltpu
from jax.experimental.pallas import tpu_sc as plsc

_NC = 2
_NS = 16
_NW = _NC * _NS
_L = 16

_MESH = plsc.VectorSubcoreMesh(core_axis_name="c", subcore_axis_name="s",
                               num_cores=_NC, num_subcores=_NS)

N = 10000
E = 160000
D = 128
P = 20000
NF = 1200
NR = 30

_NEG = -1e30


def _erf(x):
    s = jnp.sign(x)
    a = jnp.abs(x)
    t = 1.0 / (1.0 + 0.3275911 * a)
    poly = t * (0.254829592 + t * (-0.284496736 + t * (1.421413741 +
           t * (-1.453152027 + t * 1.061405429))))
    return s * (1.0 - poly * jnp.exp(-a * a))


def _gelu(x):
    return 0.5 * x * (1.0 + _erf(x * 0.7071067811865476))


def _tables_body(emb, kwn, kbn, qwn, qbn, vwn, vbn, art, mrt, mri,
                 kwe, kbe, aro, vwe, vbe, mro, awe, abe, gew, rwt,
                 scal,
                 q_tab, krt_tab, vrt_tab, kro_tab, vro_tab,
                 hidA2_tab, embB_tab, eA, eB, roleA, roleB):
    x = emb[...]
    a_e = scal[0, 0]
    ct = scal[0, 1]
    co = scal[0, 2]
    q_tab[...] = x @ qwn[...] + qbn[...]
    krt_tab[...] = (x @ (kwn[...] @ art[...]) + kbn[...] @ art[...]) * ct
    vrt_tab[...] = x @ (vwn[...] @ mrt[...]) + vbn[...] @ mrt[...]
    kro_tab[...] = (x @ (kwe[...] @ aro[...]) + kbe[...] @ aro[...]) * co
    vro_tab[...] = x @ (vwe[...] @ mro[...]) + vbe[...] @ mro[...]
    vrin = x @ (vwn[...] @ mri[...]) + vbn[...] @ mri[...]
    hidA = _gelu(vrin) @ awe[...] + abe[...]
    hidA2 = a_e * hidA
    embB = (1.0 - a_e) * x
    hidA2_tab[...] = hidA2
    embB_tab[...] = embB
    eA[...] = hidA2 @ gew[...]
    eB[...] = embB @ gew[...]
    roleA[...] = hidA2 @ rwt[...]
    roleB[...] = embB @ rwt[...]


def _make_tables(emb, prm, a_e, rwt_pad):
    bp = 2000
    grid = (P // bp,)
    scal = jnp.stack([a_e,
                      prm['p_rel_true'] / jnp.sqrt(jnp.float32(D)),
                      prm['p_rel_out'] / jnp.sqrt(jnp.float32(D))]).reshape(1, 3)

    def rep(shape):
        return pl.BlockSpec(shape, lambda i: (0,) * len(shape))

    dd = rep((D, D))
    db = rep((1, D))
    in_specs = [pl.BlockSpec((bp, D), lambda i: (i, 0)),
                dd, db, dd, db, dd, db, dd, dd, dd,
                dd, db, dd, dd, db, dd, dd, db, rep((D, 1)), rep((D, 32)),
                rep((1, 3))]
    out_specs = [pl.BlockSpec((bp, D), lambda i: (i, 0))] * 7 + \
                [pl.BlockSpec((bp, 1), lambda i: (i, 0))] * 2 + \
                [pl.BlockSpec((bp, 32), lambda i: (i, 0))] * 2
    fd = jax.ShapeDtypeStruct((P, D), jnp.float32)
    f1 = jax.ShapeDtypeStruct((P, 1), jnp.float32)
    f32s = jax.ShapeDtypeStruct((P, 32), jnp.float32)
    out_shape = [fd] * 7 + [f1, f1, f32s, f32s]
    args = (emb,
            prm['k_w_node'], prm['k_b_node'].reshape(1, D),
            prm['q_w_node'], prm['q_b_node'].reshape(1, D),
            prm['v_w_node'], prm['v_b_node'].reshape(1, D),
            prm['a_rel_true'], prm['m_rel_true'], prm['m_rel_in'],
            prm['k_w_edge'], prm['k_b_edge'].reshape(1, D),
            prm['a_rel_out'],
            prm['v_w_edge'], prm['v_b_edge'].reshape(1, D),
            prm['m_rel_out'],
            prm['a_w_edge'], prm['a_b_edge'].reshape(1, D),
            prm['gat_e_w'], rwt_pad, scal)
    return pl.pallas_call(
        _tables_body, grid=grid, in_specs=in_specs, out_specs=out_specs,
        out_shape=out_shape)(*args)


def _node_body(aggp, x_node, awn, abn, glw, grw, scal, xl, xr, gn):
    agg = aggp[0] + aggp[1]
    a_n = scal[0, 0]
    o = _gelu(agg) @ awn[...] + abn[...]
    hid = a_n * o + (1.0 - a_n) * x_node[...]
    xl[...] = hid @ glw[...] + scal[0, 1]
    xr[...] = hid @ grw[...] + scal[0, 2]
    m = jnp.max(hid, axis=0, keepdims=True)
    ez = jnp.exp(hid - m)
    gn[...] = jnp.sum(ez * hid, axis=0, keepdims=True) / jnp.sum(ez, axis=0, keepdims=True)


def _node_stage(agg_partials, x_node, prm):
    scal = jnp.stack([jax.nn.sigmoid(prm['skip_node']),
                      prm['gat_l_b'][0], prm['gat_r_b'][0]]).reshape(1, 3)
    return pl.pallas_call(
        _node_body,
        out_shape=[jax.ShapeDtypeStruct((N, 1), jnp.float32),
                   jax.ShapeDtypeStruct((N, 1), jnp.float32),
                   jax.ShapeDtypeStruct((1, D), jnp.float32)],
    )(agg_partials, x_node, prm['a_w_node'], prm['a_b_node'].reshape(1, D),
      prm['gat_l_w'], prm['gat_r_w'], scal)


def _root_body(gatp, cntp, xl, xr, scal, root_preds, amax):
    g = jnp.sum(gatp[...], axis=0)
    cnt = jnp.sum(cntp[...], axis=0, keepdims=True)
    att = scal[0, 0]
    bias = scal[0, 1]
    xlr = xl[...].reshape(1, N)
    xrr = xr[...].reshape(1, N)
    loop_eproj = g[0:1] / jnp.maximum(cnt, 1.0)
    z = xlr + xrr + loop_eproj
    s_self = jnp.maximum(z, 0.2 * z) * att
    es = jnp.exp(s_self)
    den = g[1:2] + es
    num = g[2:3] + es * xlr
    root = num / (den + 1e-16) + bias
    m = jnp.max(root, axis=1, keepdims=True)
    e = jnp.exp(root - m)
    lse = jnp.log(jnp.sum(e, axis=1, keepdims=True))
    root_preds[...] = root - m - lse
    idx = lax.broadcasted_iota(jnp.int32, (1, N), 1)
    amax[...] = jnp.min(jnp.where(root == m, idx, N), axis=1, keepdims=True)


def _root_stage(gat_partials, cnt_partials, xl, xr, prm):
    scal = jnp.stack([prm['gat_att'][0], prm['gat_bias'][0]]).reshape(1, 2)
    return pl.pallas_call(
        _root_body,
        out_shape=[jax.ShapeDtypeStruct((1, N), jnp.float32),
                   jax.ShapeDtypeStruct((1, 1), jnp.int32)],
    )(gat_partials, cnt_partials, xl, xr, scal)


def _frame_body(gn, gep, fw, fb, rwb, rb, frame, const32):
    num = jnp.sum(gep[...], axis=0)
    ge = num[0:1] / num[1:2]
    grep = jnp.concatenate([gn[...], ge], axis=1)
    f = grep @ fw[...] + fb[...]
    m = jnp.max(f, axis=1, keepdims=True)
    lse = jnp.log(jnp.sum(jnp.exp(f - m), axis=1, keepdims=True))
    frame[...] = f - m - lse
    const32[...] = gn[...] @ rwb[...] + rb[...]


def _frame_stage(gn, ge_partials, prm, rwb_pad, rb_pad):
    return pl.pallas_call(
        _frame_body,
        out_shape=[jax.ShapeDtypeStruct((1, NF), jnp.float32),
                   jax.ShapeDtypeStruct((1, 32), jnp.float32)],
    )(gn, ge_partials, prm['frame_w'], prm['frame_b'].reshape(1, NF),
      rwb_pad, rb_pad)


def _role_body(rag, rbg, src, amax, const32, out):
    r = rag[...] + rbg[...] + const32[...]
    keep = src[...] == amax[0, 0]
    r = jnp.where(keep, r, 0.0)
    lane = lax.broadcasted_iota(jnp.int32, r.shape, 1)
    valid = lane < NR
    rm = jnp.where(valid, r, _NEG)
    m = jnp.max(rm, axis=1, keepdims=True)
    e = jnp.where(valid, jnp.exp(r - m), 0.0)
    lse = jnp.log(jnp.sum(e, axis=1, keepdims=True))
    out[...] = r - m - lse


def _role_stage(rag, rbg, edge_src, amax, const32):
    be = 2000
    grid = (E // be,)
    return pl.pallas_call(
        _role_body, grid=grid,
        in_specs=[pl.BlockSpec((be, 32), lambda i: (i, 0)),
                  pl.BlockSpec((be, 32), lambda i: (i, 0)),
                  pl.BlockSpec((be, 1), lambda i: (i, 0)),
                  pl.BlockSpec((1, 1), lambda i: (0, 0)),
                  pl.BlockSpec((1, 32), lambda i: (0, 0))],
        out_specs=pl.BlockSpec((be, 32), lambda i: (i, 0)),
        out_shape=jax.ShapeDtypeStruct((E, 32), jnp.float32),
    )(rag, rbg, edge_src.reshape(E, 1), amax, const32)


def _flat_wid():
    return lax.axis_index("s") * _NC + lax.axis_index("c")


def _sc_gather_scalars(tables, idxs, chunk=640):
    np_ = len(tables)
    etot = idxs[0].shape[0]
    assert etot % chunk == 0 and chunk % _L == 0
    nch = etot // chunk

    scratch = ([pltpu.VMEM(t.shape, t.dtype) for t in tables] +
               [pltpu.VMEM((chunk,), jnp.int32) for _ in range(np_)] +
               [pltpu.VMEM((chunk,), t.dtype) for t in tables])
    out_type = [jax.ShapeDtypeStruct((etot,), t.dtype) for t in tables]

    @functools.partial(pl.kernel, out_type=out_type, mesh=_MESH,
                       scratch_types=scratch,
                       compiler_params=pltpu.CompilerParams(
                           needs_layout_passes=False,
                           use_tc_tiling_on_sc=False))
    def k(*refs):
        tab_h = refs[:np_]
        idx_h = refs[np_:2 * np_]
        out_h = refs[2 * np_:3 * np_]
        tab_v = refs[3 * np_:4 * np_]
        idx_v = refs[4 * np_:5 * np_]
        val_v = refs[5 * np_:6 * np_]
        wid = _flat_wid()
        for p in range(np_):
            pltpu.sync_copy(tab_h[p], tab_v[p])
        nloc = (nch - wid + _NW - 1) // _NW

        def body(j, _):
            base = (wid + j * _NW) * chunk
            for p in range(np_):
                pltpu.sync_copy(idx_h[p].at[pl.ds(base, chunk)], idx_v[p])
            for p in range(np_):
                for g in range(chunk // _L):
                    iv = idx_v[p][pl.ds(g * _L, _L)]
                    val_v[p][pl.ds(g * _L, _L)] = plsc.load_gather(tab_v[p], [iv])
                pltpu.sync_copy(val_v[p], out_h[p].at[pl.ds(base, chunk)])
            return 0

        lax.fori_loop(0, nloc, body, 0)

    return k(*tables, *idxs)


def _sc_gather_rows(tables, idxs, chunk=128):
    np_ = len(tables)
    etot = idxs[0].shape[0]
    assert etot % chunk == 0 and chunk % _L == 0 and chunk <= 128
    nch = etot // chunk

    scratch = ([pltpu.VMEM((chunk,), jnp.int32) for _ in range(np_)] +
               [pltpu.VMEM((chunk, t.shape[1]), t.dtype) for t in tables] +
               [pltpu.SemaphoreType.DMA])
    out_type = [jax.ShapeDtypeStruct((etot, t.shape[1]), t.dtype)
                for t in tables]

    @functools.partial(pl.kernel, out_type=out_type, mesh=_MESH,
                       scratch_types=scratch,
                       compiler_params=pltpu.CompilerParams(
                           needs_layout_passes=False,
                           use_tc_tiling_on_sc=False))
    def k(*refs):
        tab_h = refs[:np_]
        idx_h = refs[np_:2 * np_]
        out_h = refs[2 * np_:3 * np_]
        idx_v = refs[3 * np_:4 * np_]
        rows_v = refs[4 * np_:5 * np_]
        sem = refs[5 * np_]
        wid = _flat_wid()
        nloc = (nch - wid + _NW - 1) // _NW

        def body(j, _):
            base = (wid + j * _NW) * chunk
            for p in range(np_):
                pltpu.sync_copy(idx_h[p].at[pl.ds(base, chunk)], idx_v[p])
            handles = [pltpu.async_copy(tab_h[p].at[idx_v[p]], rows_v[p], sem)
                       for p in range(np_)]
            for h in handles:
                h.wait()
            for p in range(np_):
                pltpu.sync_copy(rows_v[p], out_h[p].at[pl.ds(base, chunk)])
            return 0

        lax.fori_loop(0, nloc, body, 0)

    return k(*tables, *idxs)


def _sc_scatter_scalars(vals, dst, nseg, count, chunk=640):
    nv = len(vals)
    nacc = nv + (1 if count else 0)
    etot = dst.shape[0]
    assert etot % chunk == 0 and chunk % _L == 0 and nseg % _L == 0
    nch = etot // chunk

    scratch = ([pltpu.VMEM((nseg,), jnp.float32) for _ in range(nacc)] +
               [pltpu.VMEM((chunk,), jnp.int32)] +
               [pltpu.VMEM((chunk,), jnp.float32) for _ in range(nv)])
    out_type = jax.ShapeDtypeStruct((_NW, nacc, nseg), jnp.float32)

    @functools.partial(pl.kernel, out_type=out_type, mesh=_MESH,
                       scratch_types=scratch,
                       compiler_params=pltpu.CompilerParams(
                           needs_layout_passes=False,
                           use_tc_tiling_on_sc=False))
    def k(*refs):
        val_h = refs[:nv]
        dst_h = refs[nv]
        out_h = refs[nv + 1]
        acc_v = refs[nv + 2:nv + 2 + nacc]
        dst_v = refs[nv + 2 + nacc]
        val_v = refs[nv + 3 + nacc:nv + 3 + nacc + nv]
        wid = _flat_wid()

        def zero(i, _):
            for a in acc_v:
                a[pl.ds(i * _L, _L)] = jnp.zeros((_L,), jnp.float32)
            return 0

        lax.fori_loop(0, nseg // _L, zero, 0)
        nloc = (nch - wid + _NW - 1) // _NW

        def body(j, _):
            base = (wid + j * _NW) * chunk
            pltpu.sync_copy(dst_h.at[pl.ds(base, chunk)], dst_v)
            for p in range(nv):
                pltpu.sync_copy(val_h[p].at[pl.ds(base, chunk)], val_v[p])
            for g in range(chunk // _L):
                dv = dst_v[pl.ds(g * _L, _L)]
                for p in range(nv):
                    plsc.addupdate_scatter(acc_v[p], [dv],
                                           val_v[p][pl.ds(g * _L, _L)])
                if count:
                    plsc.addupdate_scatter(acc_v[nv], [dv],
                                           jnp.ones((_L,), jnp.float32))
            return 0

        lax.fori_loop(0, nloc, body, 0)
        for p in range(nacc):
            pltpu.sync_copy(acc_v[p], out_h.at[wid, p])

    return k(*vals, dst)


def _sc_scatter_rows(rows, dst, zeros, nseg, chunk=128):
    etot, w = rows.shape
    assert etot % chunk == 0 and chunk <= 128 and nseg % _NS == 0
    nch = etot // chunk
    rows_per_tile = nseg // _NS

    scratch = [pltpu.VMEM_SHARED((nseg, w), jnp.float32),
               pltpu.VMEM((chunk,), jnp.int32),
               pltpu.VMEM((chunk, w), jnp.float32)]
    out_type = jax.ShapeDtypeStruct((_NC, nseg, w), jnp.float32)

    @functools.partial(pl.kernel, out_type=out_type, mesh=_MESH,
                       scratch_types=scratch,
                       compiler_params=pltpu.CompilerParams(
                           needs_layout_passes=False,
                           use_tc_tiling_on_sc=False))
    def k(rows_h, dst_h, zeros_h, out_h, acc_s, dst_v, rows_v):
        cid = lax.axis_index("c")
        sid = lax.axis_index("s")
        wid = sid * _NC + cid
        row0 = sid * rows_per_tile
        pltpu.sync_copy(zeros_h.at[pl.ds(row0, rows_per_tile)],
                        acc_s.at[pl.ds(row0, rows_per_tile)])
        plsc.subcore_barrier()
        nloc = (nch - wid + _NW - 1) // _NW

        def body(j, _):
            base = (wid + j * _NW) * chunk
            pltpu.sync_copy(dst_h.at[pl.ds(base, chunk)], dst_v)
            pltpu.sync_copy(rows_h.at[pl.ds(base, chunk)], rows_v)
            pltpu.sync_copy(rows_v, acc_s.at[dst_v], add=True)
            return 0

        lax.fori_loop(0, nloc, body, 0)
        plsc.subcore_barrier()
        pltpu.sync_copy(acc_s.at[pl.ds(row0, rows_per_tile)],
                        out_h.at[cid, pl.ds(row0, rows_per_tile)])

    return k(rows, dst, zeros)


def _scores_body(qg, krtg, krog, et, eo):
    q = qg[...]
    et[...] = jnp.exp(jnp.sum(krtg[...] * q, axis=1, keepdims=True))
    eo[...] = jnp.exp(jnp.sum(krog[...] * q, axis=1, keepdims=True))


def _scores_stage(qg, krtg, krog):
    be = 2000
    return pl.pallas_call(
        _scores_body, grid=(E // be,),
        in_specs=[pl.BlockSpec((be, D), lambda i: (i, 0))] * 3,
        out_specs=[pl.BlockSpec((be, 1), lambda i: (i, 0))] * 2,
        out_shape=[jax.ShapeDtypeStruct((E, 1), jnp.float32)] * 2,
    )(qg, krtg, krog)


def _combine_body(pp, out):
    out[...] = jnp.sum(pp[...], axis=0)


def _combine_stage(partials):
    k, na, n = partials.shape
    return pl.pallas_call(
        _combine_body,
        out_shape=jax.ShapeDtypeStruct((na, n), jnp.float32),
    )(partials)


def _comb_rows_body(vrtg, vrog, et, eo, dtg, dog, comb):
    at = et[...] / (dtg[...] + 1e-16)
    ao = eo[...] / (dog[...] + 1e-16)
    comb[...] = at * vrtg[...] + ao * vrog[...]


def _comb_rows_stage(vrtg, vrog, et, eo, dtg, dog):
    be = 2000
    return pl.pallas_call(
        _comb_rows_body, grid=(E // be,),
        in_specs=[pl.BlockSpec((be, D), lambda i: (i, 0))] * 2 +
                 [pl.BlockSpec((be, 1), lambda i: (i, 0))] * 4,
        out_specs=pl.BlockSpec((be, D), lambda i: (i, 0)),
        out_shape=jax.ShapeDtypeStruct((E, D), jnp.float32),
    )(vrtg, vrog, et, eo, dtg.reshape(E, 1), dog.reshape(E, 1))


def _softagg_body(ha, eb, out):
    z = ha[...] + eb[...]
    ez = jnp.exp(z)
    nm = jnp.sum(ez * z, axis=0, keepdims=True)
    dn = jnp.sum(ez, axis=0, keepdims=True)
    blk = jnp.concatenate([nm, dn], axis=0).reshape(1, 2, D)

    @pl.when(pl.program_id(0) == 0)
    def _():
        out[...] = blk

    @pl.when(pl.program_id(0) != 0)
    def _():
        out[...] += blk


def _softagg_stage(hA2g, embBg):
    be = 2000
    return pl.pallas_call(
        _softagg_body, grid=(E // be,),
        in_specs=[pl.BlockSpec((be, D), lambda i: (i, 0))] * 2,
        out_specs=pl.BlockSpec((1, 2, D), lambda i: (0, 0, 0)),
        out_shape=jax.ShapeDtypeStruct((1, 2, D), jnp.float32),
    )(hA2g, embBg)


def _gat_edge_body(xls, xrd, eag, ebg, scal, eproj, es, esx):
    ep = eag[...] + ebg[...]
    z = xls[...] + xrd[...] + ep
    s = jnp.maximum(z, 0.2 * z) * scal[0, 0]
    e = jnp.exp(s)
    eproj[...] = ep
    es[...] = e
    esx[...] = e * xls[...]


def _gat_edge_stage(xls, xrd, eag, ebg, prm):
    be = 2000
    scal = prm['gat_att'].reshape(1, 1)
    return pl.pallas_call(
        _gat_edge_body, grid=(E // be,),
        in_specs=[pl.BlockSpec((be, 1), lambda i: (i, 0))] * 4 +
                 [pl.BlockSpec((1, 1), lambda i: (0, 0))],
        out_specs=[pl.BlockSpec((be, 1), lambda i: (i, 0))] * 3,
        out_shape=[jax.ShapeDtypeStruct((E, 1), jnp.float32)] * 3,
    )(xls.reshape(E, 1), xrd.reshape(E, 1), eag.reshape(E, 1),
      ebg.reshape(E, 1), scal)


def kernel(node_x, edge_x, edge_src, edge_dst, params):
    prm = params
    emb = prm['pred_emb']
    a_e = jax.nn.sigmoid(prm['skip_edge'])
    rwt_pad = jnp.pad(prm['role_w'][:D], ((0, 0), (0, 32 - NR)))
    rwb_pad = jnp.pad(prm['role_w'][D:], ((0, 0), (0, 32 - NR)))
    rb_pad = jnp.pad(prm['role_b'], (0, 32 - NR)).reshape(1, 32)

    (q_tab, krt_tab, vrt_tab, kro_tab, vro_tab,
     hidA2_tab, embB_tab, eA, eB, roleA, roleB) = _make_tables(emb, prm, a_e, rwt_pad)

    cs, cd = _sc_gather_scalars([node_x, node_x], [edge_src, edge_dst])
    ce = edge_x
    (x_node,) = _sc_gather_rows([emb], [node_x], chunk=80)

    qg, krtg, krog, hA2g, eBg, rAg, rBg = _sc_gather_rows(
        [q_tab, krt_tab, kro_tab, hidA2_tab, embB_tab, roleA, roleB],
        [cd, cs, ce, cs, ce, cs, ce])

    et2, eo2 = _scores_stage(qg, krtg, krog)
    e_t = et2.reshape(E)
    e_o = eo2.reshape(E)
    den_partials = _sc_scatter_scalars([e_t, e_o], edge_dst, N, count=True)
    dens = _combine_stage(den_partials)
    den_t = dens[0]
    den_o = dens[1]
    cnt_partials = dens[2:3]

    dtg, dog = _sc_gather_scalars([den_t, den_o], [edge_dst, edge_dst])
    vrtg, vrog = _sc_gather_rows([vrt_tab, vro_tab], [cs, ce])
    comb = _comb_rows_stage(vrtg, vrog, et2, eo2, dtg, dog)
    zeros_nd = jnp.zeros((N, D), jnp.float32)
    agg_partials = _sc_scatter_rows(comb, edge_dst, zeros_nd, N)

    ge_partials = _softagg_stage(hA2g, eBg)

    xl, xr, gn = _node_stage(agg_partials, x_node, prm)

    xls, xrd, eag, ebg = _sc_gather_scalars(
        [xl.reshape(N), xr.reshape(N), eA.reshape(P), eB.reshape(P)],
        [edge_src, edge_dst, cs, ce])
    eproj2, es2, esx2 = _gat_edge_stage(xls, xrd, eag, ebg, prm)
    gat_partials = _sc_scatter_scalars(
        [eproj2.reshape(E), es2.reshape(E), esx2.reshape(E)],
        edge_dst, N, count=False)

    root_preds2, amax = _root_stage(gat_partials, cnt_partials, xl, xr, prm)
    frame2, const32 = _frame_stage(gn, ge_partials, prm, rwb_pad, rb_pad)

    role32 = _role_stage(rAg, rBg, edge_src, amax, const32)

    root_preds = root_preds2.reshape(N)
    frame_preds = frame2.reshape(NF)
    role_preds = role32[:, :NR]
    return ((root_preds, frame_preds), role_preds)
